# baseline (speedup 1.0000x reference)
import functools

import jax
import jax.numpy as jnp
from jax import lax
from jax.experimental import pallas as pl
from jax.experimental.pallas import tpu as pltpu
from jax.experimental.pallas import tpu_sc as plsc

L = 16
NC, NS = 2, 16
NW = NC * NS
VP = 2 * L


def _make_kernel(B, SEQ, EMB1):
    rows = B // NW
    chunks = rows // L
    TOK = SEQ * rows
    TTL = EMB1 * VP
    WBL = EMB1 * L

    mesh = plsc.VectorSubcoreMesh(core_axis_name="c", subcore_axis_name="s")

    @functools.partial(
        pl.kernel,
        out_type=jax.ShapeDtypeStruct((B,), jnp.float32),
        mesh=mesh,
        compiler_params=pltpu.CompilerParams(needs_layout_passes=False),
        scratch_types=[
            pltpu.VMEM((TOK,), jnp.int32),
            pltpu.VMEM((TTL,), jnp.int32),
            pltpu.VMEM((WBL,), jnp.int32),
            pltpu.VMEM((VP,), jnp.float32),
            pltpu.VMEM((rows,), jnp.float32),
            pltpu.SemaphoreType.DMA,
            pltpu.SemaphoreType.DMA,
        ],
    )
    def sc_kernel(packed_hbm, out_hbm,
                  tok_v, tt_v, wb_v, lut_v, out_v, tsem, wsem):
        wid = lax.axis_index("s") * NC + lax.axis_index("c")

        tok_cp = pltpu.async_copy(packed_hbm.at[pl.ds(wid * TOK, TOK)],
                                  tok_v, tsem)
        tt_cp = pltpu.async_copy(packed_hbm.at[pl.ds(NW * TOK, TTL)],
                                 tt_v, wsem)
        wb_cp = pltpu.async_copy(packed_hbm.at[pl.ds(NW * TOK + TTL, WBL)],
                                 wb_v, wsem)
        tt_cp.wait()
        wb_cp.wait()

        def lut_body(d, acc):
            a0, a1 = acc
            w = plsc.bitcast(wb_v[pl.ds(d * L, L)], jnp.float32)
            t0 = plsc.bitcast(tt_v[pl.ds(d * VP, L)], jnp.float32)
            t1 = plsc.bitcast(tt_v[pl.ds(d * VP + L, L)], jnp.float32)
            return (a0 + t0 * w, a1 + t1 * w)

        a0, a1 = lax.fori_loop(
            0, EMB1, lut_body,
            (jnp.zeros((L,), jnp.float32), jnp.zeros((L,), jnp.float32)))
        inv_seq = jnp.float32(1.0 / SEQ)
        lut_v[pl.ds(0, L)] = a0 * inv_seq
        lut_v[pl.ds(L, L)] = a1 * inv_seq

        tok_cp.wait()

        def row_body(j, carry):
            g = None
            for s in range(SEQ):
                t = tok_v[pl.ds(s * rows + j * L, L)]
                gs = plsc.load_gather(lut_v, [t])
                g = gs if g is None else g + gs
            out_v[pl.ds(j * L, L)] = 1.0 / (1.0 + jnp.exp(-g))
            return carry

        lax.fori_loop(0, chunks, row_body, jnp.int32(0))

        pltpu.sync_copy(out_v, out_hbm.at[pl.ds(wid * rows, rows)])

    return sc_kernel


def kernel(tokens, table, W, b):
    B, SEQ = tokens.shape
    VOCAB, EMB = table.shape
    rows = B // NW

    tok_flat = tokens.reshape(NW, rows, SEQ).transpose(0, 2, 1).reshape(-1)
    tt = jnp.pad(table.T, ((0, 1), (0, VP - VOCAB)),
                 constant_values=1.0).reshape(-1)
    wb = jnp.concatenate([W.reshape(EMB), b]).reshape(EMB + 1, 1)
    wb = jnp.broadcast_to(wb, (EMB + 1, L)).reshape(-1)
    packed = jnp.concatenate([
        tok_flat,
        lax.bitcast_convert_type(tt, jnp.int32),
        lax.bitcast_convert_type(wb, jnp.int32),
    ])

    out = _make_kernel(B, SEQ, EMB + 1)(packed)
    return out.reshape(B, 1)


# speedup vs baseline: 1.1379x; 1.1379x over previous
import functools

import jax
import jax.numpy as jnp
from jax import lax
from jax.experimental import pallas as pl
from jax.experimental.pallas import tpu as pltpu
from jax.experimental.pallas import tpu_sc as plsc

L = 16
NC, NS = 2, 16
NW = NC * NS
VP = 2 * L


def _make_kernel(B, SEQ, EMB1):
    rows = B // NW
    chunks = rows // L

    mesh = plsc.VectorSubcoreMesh(core_axis_name="c", subcore_axis_name="s")

    @functools.partial(
        pl.kernel,
        out_type=jax.ShapeDtypeStruct((B,), jnp.float32),
        mesh=mesh,
        compiler_params=pltpu.CompilerParams(needs_layout_passes=False),
        scratch_types=[
            pltpu.VMEM((SEQ * rows,), jnp.int32),
            pltpu.VMEM((EMB1 * VP,), jnp.float32),
            pltpu.VMEM((EMB1 * L,), jnp.float32),
            pltpu.VMEM((VP,), jnp.float32),
            pltpu.VMEM((rows,), jnp.float32),
            pltpu.SemaphoreType.DMA,
            pltpu.SemaphoreType.DMA,
        ],
    )
    def sc_kernel(tok_hbm, tt_hbm, wb_hbm, out_hbm,
                  tok_v, tt_v, wb_v, lut_v, out_v, tsem, wsem):
        wid = lax.axis_index("s") * NC + lax.axis_index("c")

        tok_cp = pltpu.async_copy(tok_hbm.at[wid], tok_v, tsem)
        tt_cp = pltpu.async_copy(tt_hbm, tt_v, wsem)
        wb_cp = pltpu.async_copy(wb_hbm, wb_v, wsem)
        tt_cp.wait()
        wb_cp.wait()

        def lut_body(i, acc):
            p0, p1, q0, q1 = acc
            d = i * 2
            w0 = wb_v[pl.ds(d * L, L)]
            w1 = wb_v[pl.ds(d * L + L, L)]
            p0 = p0 + tt_v[pl.ds(d * VP, L)] * w0
            p1 = p1 + tt_v[pl.ds(d * VP + L, L)] * w0
            q0 = q0 + tt_v[pl.ds(d * VP + VP, L)] * w1
            q1 = q1 + tt_v[pl.ds(d * VP + VP + L, L)] * w1
            return (p0, p1, q0, q1)

        z = jnp.zeros((L,), jnp.float32)
        p0, p1, q0, q1 = lax.fori_loop(0, EMB1 // 2, lut_body, (z, z, z, z))
        a0, a1 = p0 + q0, p1 + q1
        if EMB1 % 2:
            d = EMB1 - 1
            w = wb_v[pl.ds(d * L, L)]
            a0 = a0 + tt_v[pl.ds(d * VP, L)] * w
            a1 = a1 + tt_v[pl.ds(d * VP + L, L)] * w
        inv_seq = jnp.float32(1.0 / SEQ)
        lut_v[pl.ds(0, L)] = a0 * inv_seq
        lut_v[pl.ds(L, L)] = a1 * inv_seq

        tok_cp.wait()

        def row_body(i, carry):
            j = i * 2
            ga = None
            gb = None
            for s in range(SEQ):
                ta = tok_v[pl.ds(s * rows + j * L, L)]
                tb = tok_v[pl.ds(s * rows + j * L + L, L)]
                gsa = plsc.load_gather(lut_v, [ta])
                gsb = plsc.load_gather(lut_v, [tb])
                ga = gsa if ga is None else ga + gsa
                gb = gsb if gb is None else gb + gsb
            out_v[pl.ds(j * L, L)] = 1.0 / (1.0 + jnp.exp(-ga))
            out_v[pl.ds(j * L + L, L)] = 1.0 / (1.0 + jnp.exp(-gb))
            return carry

        lax.fori_loop(0, chunks // 2, row_body, jnp.int32(0))

        pltpu.sync_copy(out_v, out_hbm.at[pl.ds(wid * rows, rows)])

    return sc_kernel


def kernel(tokens, table, W, b):
    B, SEQ = tokens.shape
    VOCAB, EMB = table.shape
    rows = B // NW

    tok_w = tokens.reshape(NW, rows, SEQ).transpose(0, 2, 1).reshape(NW, SEQ * rows)
    tt = jnp.pad(table.T, ((0, 1), (0, VP - VOCAB)),
                 constant_values=1.0).reshape(-1)
    wb = jnp.concatenate([W.reshape(EMB), b]).reshape(EMB + 1, 1)
    wb = jnp.broadcast_to(wb, (EMB + 1, L)).reshape(-1)

    out = _make_kernel(B, SEQ, EMB + 1)(tok_w, tt, wb)
    return out.reshape(B, 1)


# speedup vs baseline: 1.1440x; 1.0053x over previous
import functools

import jax
import jax.numpy as jnp
from jax import lax
from jax.experimental import pallas as pl
from jax.experimental.pallas import tpu as pltpu
from jax.experimental.pallas import tpu_sc as plsc

L = 16
NC, NS = 2, 16
NW = NC * NS
VP = 2 * L


def _make_kernel(B, SEQ, EMB1, bits):
    rows = B // NW
    chunks = rows // L
    mask = (1 << bits) - 1

    mesh = plsc.VectorSubcoreMesh(core_axis_name="c", subcore_axis_name="s")

    @functools.partial(
        pl.kernel,
        out_type=jax.ShapeDtypeStruct((B,), jnp.float32),
        mesh=mesh,
        compiler_params=pltpu.CompilerParams(needs_layout_passes=False),
        scratch_types=[
            pltpu.VMEM((rows,), jnp.int32),
            pltpu.VMEM((EMB1 * VP,), jnp.float32),
            pltpu.VMEM((EMB1 * L,), jnp.float32),
            pltpu.VMEM((VP,), jnp.float32),
            pltpu.VMEM((rows,), jnp.float32),
            pltpu.SemaphoreType.DMA,
            pltpu.SemaphoreType.DMA,
        ],
    )
    def sc_kernel(tok_hbm, tt_hbm, wb_hbm, out_hbm,
                  tok_v, tt_v, wb_v, lut_v, out_v, tsem, wsem):
        wid = lax.axis_index("s") * NC + lax.axis_index("c")

        tok_cp = pltpu.async_copy(tok_hbm.at[pl.ds(wid * rows, rows)],
                                  tok_v, tsem)
        tt_cp = pltpu.async_copy(tt_hbm, tt_v, wsem)
        wb_cp = pltpu.async_copy(wb_hbm, wb_v, wsem)
        tt_cp.wait()
        wb_cp.wait()

        def lut_body(i, acc):
            p0, p1, q0, q1 = acc
            d = i * 2
            w0 = wb_v[pl.ds(d * L, L)]
            w1 = wb_v[pl.ds(d * L + L, L)]
            p0 = p0 + tt_v[pl.ds(d * VP, L)] * w0
            p1 = p1 + tt_v[pl.ds(d * VP + L, L)] * w0
            q0 = q0 + tt_v[pl.ds(d * VP + VP, L)] * w1
            q1 = q1 + tt_v[pl.ds(d * VP + VP + L, L)] * w1
            return (p0, p1, q0, q1)

        z = jnp.zeros((L,), jnp.float32)
        p0, p1, q0, q1 = lax.fori_loop(0, EMB1 // 2, lut_body, (z, z, z, z))
        a0, a1 = p0 + q0, p1 + q1
        if EMB1 % 2:
            d = EMB1 - 1
            w = wb_v[pl.ds(d * L, L)]
            a0 = a0 + tt_v[pl.ds(d * VP, L)] * w
            a1 = a1 + tt_v[pl.ds(d * VP + L, L)] * w
        inv_seq = jnp.float32(1.0 / SEQ)
        lut_v[pl.ds(0, L)] = a0 * inv_seq
        lut_v[pl.ds(L, L)] = a1 * inv_seq

        tok_cp.wait()

        def row_body(i, carry):
            j = i * 2
            ta = tok_v[pl.ds(j * L, L)]
            tb = tok_v[pl.ds(j * L + L, L)]
            ga = None
            gb = None
            for s in range(SEQ):
                ia = (ta >> (bits * s)) & mask
                ib = (tb >> (bits * s)) & mask
                gsa = plsc.load_gather(lut_v, [ia])
                gsb = plsc.load_gather(lut_v, [ib])
                ga = gsa if ga is None else ga + gsa
                gb = gsb if gb is None else gb + gsb
            out_v[pl.ds(j * L, L)] = 1.0 / (1.0 + jnp.exp(-ga))
            out_v[pl.ds(j * L + L, L)] = 1.0 / (1.0 + jnp.exp(-gb))
            return carry

        lax.fori_loop(0, chunks // 2, row_body, jnp.int32(0))

        pltpu.sync_copy(out_v, out_hbm.at[pl.ds(wid * rows, rows)])

    return sc_kernel


def kernel(tokens, table, W, b):
    B, SEQ = tokens.shape
    VOCAB, EMB = table.shape

    bits = max((VOCAB - 1).bit_length(), 1)
    shifts = jnp.asarray([1 << (bits * s) for s in range(SEQ)], jnp.int32)
    tok_packed = (tokens * shifts).sum(axis=1, dtype=jnp.int32)
    tt = jnp.pad(table.T, ((0, 1), (0, VP - VOCAB)),
                 constant_values=1.0).reshape(-1)
    wb = jnp.concatenate([W.reshape(EMB), b]).reshape(EMB + 1, 1)
    wb = jnp.broadcast_to(wb, (EMB + 1, L)).reshape(-1)

    out = _make_kernel(B, SEQ, EMB + 1, bits)(tok_packed, tt, wb)
    return out.reshape(B, 1)


# speedup vs baseline: 1.1459x; 1.0017x over previous
import functools

import jax
import jax.numpy as jnp
from jax import lax
from jax.experimental import pallas as pl
from jax.experimental.pallas import tpu as pltpu
from jax.experimental.pallas import tpu_sc as plsc

L = 16
NC, NS = 2, 16
NW = NC * NS
VP = 2 * L


def _make_kernel(B, SEQ, EMB1, bits):
    rows = B // NW
    chunks = rows // L
    mask = (1 << bits) - 1

    mesh = plsc.VectorSubcoreMesh(core_axis_name="c", subcore_axis_name="s")

    @functools.partial(
        pl.kernel,
        out_type=jax.ShapeDtypeStruct((B,), jnp.float32),
        mesh=mesh,
        compiler_params=pltpu.CompilerParams(needs_layout_passes=False),
        scratch_types=[
            pltpu.VMEM((rows,), jnp.int32),
            pltpu.VMEM((EMB1 * VP,), jnp.float32),
            pltpu.VMEM((EMB1 * L,), jnp.float32),
            pltpu.VMEM((VP,), jnp.float32),
            pltpu.VMEM((rows,), jnp.float32),
            pltpu.SemaphoreType.DMA,
            pltpu.SemaphoreType.DMA,
        ],
    )
    def sc_kernel(tok_hbm, tt_hbm, wb_hbm, out_hbm,
                  tok_v, tt_v, wb_v, lut_v, out_v, tsem, wsem):
        wid = lax.axis_index("s") * NC + lax.axis_index("c")

        tok_cp = pltpu.async_copy(tok_hbm.at[pl.ds(wid * rows, rows)],
                                  tok_v, tsem)
        tt_cp = pltpu.async_copy(tt_hbm, tt_v, wsem)
        wb_cp = pltpu.async_copy(wb_hbm, wb_v, wsem)
        tt_cp.wait()
        wb_cp.wait()

        def lut_body(i, acc):
            p0, p1, q0, q1 = acc
            d = i * 2
            w0 = wb_v[pl.ds(d * L, L)]
            w1 = wb_v[pl.ds(d * L + L, L)]
            p0 = p0 + tt_v[pl.ds(d * VP, L)] * w0
            p1 = p1 + tt_v[pl.ds(d * VP + L, L)] * w0
            q0 = q0 + tt_v[pl.ds(d * VP + VP, L)] * w1
            q1 = q1 + tt_v[pl.ds(d * VP + VP + L, L)] * w1
            return (p0, p1, q0, q1)

        z = jnp.zeros((L,), jnp.float32)
        p0, p1, q0, q1 = lax.fori_loop(0, EMB1 // 2, lut_body, (z, z, z, z))
        a0, a1 = p0 + q0, p1 + q1
        if EMB1 % 2:
            d = EMB1 - 1
            w = wb_v[pl.ds(d * L, L)]
            a0 = a0 + tt_v[pl.ds(d * VP, L)] * w
            a1 = a1 + tt_v[pl.ds(d * VP + L, L)] * w
        inv_seq = jnp.float32(1.0 / SEQ)
        lut_v[pl.ds(0, L)] = a0 * inv_seq
        lut_v[pl.ds(L, L)] = a1 * inv_seq

        tok_cp.wait()

        def row_body(i, carry):
            t4 = [tok_v[pl.ds((i * 4 + k) * L, L)] for k in range(4)]
            g4 = [None] * 4
            for s in range(SEQ):
                for k in range(4):
                    idx = (t4[k] >> (bits * s)) & mask
                    gs = plsc.load_gather(lut_v, [idx])
                    g4[k] = gs if g4[k] is None else g4[k] + gs
            for k in range(4):
                out_v[pl.ds((i * 4 + k) * L, L)] = 1.0 / (1.0 + jnp.exp(-g4[k]))
            return carry

        half = rows // 2
        lax.fori_loop(0, chunks // 8, row_body, jnp.int32(0))
        out_cp = pltpu.async_copy(out_v.at[pl.ds(0, half)],
                                  out_hbm.at[pl.ds(wid * rows, half)], tsem)
        lax.fori_loop(chunks // 8, chunks // 4, row_body, jnp.int32(0))
        pltpu.sync_copy(out_v.at[pl.ds(half, half)],
                        out_hbm.at[pl.ds(wid * rows + half, half)])
        out_cp.wait()

    return sc_kernel


def kernel(tokens, table, W, b):
    B, SEQ = tokens.shape
    VOCAB, EMB = table.shape

    bits = max((VOCAB - 1).bit_length(), 1)
    shifts = jnp.asarray([1 << (bits * s) for s in range(SEQ)], jnp.int32)
    tok_packed = (tokens * shifts).sum(axis=1, dtype=jnp.int32)
    tt = jnp.pad(table.T, ((0, 1), (0, VP - VOCAB)),
                 constant_values=1.0).reshape(-1)
    wb = jnp.concatenate([W.reshape(EMB), b]).reshape(EMB + 1, 1)
    wb = jnp.broadcast_to(wb, (EMB + 1, L)).reshape(-1)

    out = _make_kernel(B, SEQ, EMB + 1, bits)(tok_packed, tt, wb)
    return out.reshape(B, 1)


# speedup vs baseline: 1.1985x; 1.0459x over previous
import functools

import jax
import jax.numpy as jnp
from jax import lax
from jax.experimental import pallas as pl
from jax.experimental.pallas import tpu as pltpu
from jax.experimental.pallas import tpu_sc as plsc

L = 16
NC, NS = 2, 16
NW = NC * NS
VP = 2 * L


def _make_kernel(B, SEQ, EMB1, bits):
    rows = B // NW
    chunks = rows // L
    mask = (1 << bits) - 1
    ROW = VP + L

    mesh = plsc.VectorSubcoreMesh(core_axis_name="c", subcore_axis_name="s")

    @functools.partial(
        pl.kernel,
        out_type=jax.ShapeDtypeStruct((B,), jnp.float32),
        mesh=mesh,
        compiler_params=pltpu.CompilerParams(needs_layout_passes=False),
        scratch_types=[
            pltpu.VMEM((rows,), jnp.int32),
            pltpu.VMEM((EMB1 * ROW,), jnp.float32),
            pltpu.VMEM((VP,), jnp.float32),
            pltpu.VMEM((rows,), jnp.float32),
            pltpu.SemaphoreType.DMA,
            pltpu.SemaphoreType.DMA,
        ],
    )
    def sc_kernel(tok_hbm, wt_hbm, out_hbm,
                  tok_v, wt_v, lut_v, out_v, tsem, wsem):
        wid = lax.axis_index("s") * NC + lax.axis_index("c")

        tok_cp = pltpu.async_copy(tok_hbm.at[pl.ds(wid * rows, rows)],
                                  tok_v, tsem)
        wt_cp = pltpu.async_copy(wt_hbm, wt_v, wsem)
        wt_cp.wait()

        def lut_body(i, acc):
            p0, p1, q0, q1 = acc
            d = i * 2
            w0 = wt_v[pl.ds(d * ROW + VP, L)]
            w1 = wt_v[pl.ds(d * ROW + ROW + VP, L)]
            p0 = p0 + wt_v[pl.ds(d * ROW, L)] * w0
            p1 = p1 + wt_v[pl.ds(d * ROW + L, L)] * w0
            q0 = q0 + wt_v[pl.ds(d * ROW + ROW, L)] * w1
            q1 = q1 + wt_v[pl.ds(d * ROW + ROW + L, L)] * w1
            return (p0, p1, q0, q1)

        z = jnp.zeros((L,), jnp.float32)
        p0, p1, q0, q1 = lax.fori_loop(0, EMB1 // 2, lut_body, (z, z, z, z))
        a0, a1 = p0 + q0, p1 + q1
        if EMB1 % 2:
            d = EMB1 - 1
            w = wt_v[pl.ds(d * ROW + VP, L)]
            a0 = a0 + wt_v[pl.ds(d * ROW, L)] * w
            a1 = a1 + wt_v[pl.ds(d * ROW + L, L)] * w
        inv_seq = jnp.float32(1.0 / SEQ)
        lut_v[pl.ds(0, L)] = a0 * inv_seq
        lut_v[pl.ds(L, L)] = a1 * inv_seq

        tok_cp.wait()

        def row_body(i, carry):
            t4 = [tok_v[pl.ds((i * 4 + k) * L, L)] for k in range(4)]
            g4 = [None] * 4
            for s in range(SEQ):
                for k in range(4):
                    idx = (t4[k] >> (bits * s)) & mask
                    gs = plsc.load_gather(lut_v, [idx])
                    g4[k] = gs if g4[k] is None else g4[k] + gs
            for k in range(4):
                out_v[pl.ds((i * 4 + k) * L, L)] = 1.0 / (1.0 + jnp.exp(-g4[k]))
            return carry

        half = rows // 2
        lax.fori_loop(0, chunks // 8, row_body, jnp.int32(0))
        out_cp = pltpu.async_copy(out_v.at[pl.ds(0, half)],
                                  out_hbm.at[pl.ds(wid * rows, half)], tsem)
        lax.fori_loop(chunks // 8, chunks // 4, row_body, jnp.int32(0))
        pltpu.sync_copy(out_v.at[pl.ds(half, half)],
                        out_hbm.at[pl.ds(wid * rows + half, half)])
        out_cp.wait()

    return sc_kernel


def kernel(tokens, table, W, b):
    B, SEQ = tokens.shape
    VOCAB, EMB = table.shape

    bits = max((VOCAB - 1).bit_length(), 1)
    shifts = jnp.asarray([1 << (bits * s) for s in range(SEQ)], jnp.int32)
    tok_packed = (tokens * shifts).sum(axis=1, dtype=jnp.int32)
    tt = jnp.pad(table.T, ((0, 1), (0, VP - VOCAB)),
                 constant_values=1.0)
    wb = jnp.concatenate([W.reshape(EMB), b]).reshape(EMB + 1, 1)
    wb = jnp.broadcast_to(wb, (EMB + 1, L))
    wt = jnp.concatenate([tt, wb], axis=1).reshape(-1)

    out = _make_kernel(B, SEQ, EMB + 1, bits)(tok_packed, wt)
    return out.reshape(B, 1)


# speedup vs baseline: 1.2015x; 1.0026x over previous
import functools

import jax
import jax.numpy as jnp
from jax import lax
from jax.experimental import pallas as pl
from jax.experimental.pallas import tpu as pltpu
from jax.experimental.pallas import tpu_sc as plsc

L = 16
NC, NS = 2, 16
NW = NC * NS
VP = 2 * L


def _make_kernel(B, SEQ, EMB1, bits):
    rows = B // NW
    chunks = rows // L
    mask = (1 << bits) - 1
    ROW = VP + L

    mesh = plsc.VectorSubcoreMesh(core_axis_name="c", subcore_axis_name="s")

    @functools.partial(
        pl.kernel,
        out_type=jax.ShapeDtypeStruct((B,), jnp.float32),
        mesh=mesh,
        compiler_params=pltpu.CompilerParams(needs_layout_passes=False),
        scratch_types=[
            pltpu.VMEM((rows,), jnp.int32),
            pltpu.VMEM((EMB1 * ROW,), jnp.int32),
            pltpu.VMEM((VP,), jnp.float32),
            pltpu.VMEM((rows,), jnp.float32),
            pltpu.SemaphoreType.DMA,
            pltpu.SemaphoreType.DMA,
        ],
    )
    def sc_kernel(packed_hbm, out_hbm,
                  tok_v, wt_v, lut_v, out_v, tsem, wsem):
        wid = lax.axis_index("s") * NC + lax.axis_index("c")

        tok_cp = pltpu.async_copy(packed_hbm.at[pl.ds(wid * rows, rows)],
                                  tok_v, tsem)
        wt_cp = pltpu.async_copy(packed_hbm.at[pl.ds(B, EMB1 * ROW)],
                                 wt_v, wsem)
        wt_cp.wait()

        def wld(off):
            return plsc.bitcast(wt_v[pl.ds(off, L)], jnp.float32)

        def lut_body(i, acc):
            p0, p1, q0, q1 = acc
            d = i * 2
            w0 = wld(d * ROW + VP)
            w1 = wld(d * ROW + ROW + VP)
            p0 = p0 + wld(d * ROW) * w0
            p1 = p1 + wld(d * ROW + L) * w0
            q0 = q0 + wld(d * ROW + ROW) * w1
            q1 = q1 + wld(d * ROW + ROW + L) * w1
            return (p0, p1, q0, q1)

        z = jnp.zeros((L,), jnp.float32)
        p0, p1, q0, q1 = lax.fori_loop(0, EMB1 // 2, lut_body, (z, z, z, z))
        a0, a1 = p0 + q0, p1 + q1
        if EMB1 % 2:
            d = EMB1 - 1
            w = wld(d * ROW + VP)
            a0 = a0 + wld(d * ROW) * w
            a1 = a1 + wld(d * ROW + L) * w
        inv_seq = jnp.float32(1.0 / SEQ)
        lut_v[pl.ds(0, L)] = a0 * inv_seq
        lut_v[pl.ds(L, L)] = a1 * inv_seq

        tok_cp.wait()

        def row_body(i, carry):
            t4 = [tok_v[pl.ds((i * 4 + k) * L, L)] for k in range(4)]
            g4 = [None] * 4
            for s in range(SEQ):
                for k in range(4):
                    idx = (t4[k] >> (bits * s)) & mask
                    gs = plsc.load_gather(lut_v, [idx])
                    g4[k] = gs if g4[k] is None else g4[k] + gs
            for k in range(4):
                out_v[pl.ds((i * 4 + k) * L, L)] = 1.0 / (1.0 + jnp.exp(-g4[k]))
            return carry

        half = rows // 2
        lax.fori_loop(0, chunks // 8, row_body, jnp.int32(0))
        out_cp = pltpu.async_copy(out_v.at[pl.ds(0, half)],
                                  out_hbm.at[pl.ds(wid * rows, half)], tsem)
        lax.fori_loop(chunks // 8, chunks // 4, row_body, jnp.int32(0))
        pltpu.sync_copy(out_v.at[pl.ds(half, half)],
                        out_hbm.at[pl.ds(wid * rows + half, half)])
        out_cp.wait()

    return sc_kernel


def kernel(tokens, table, W, b):
    B, SEQ = tokens.shape
    VOCAB, EMB = table.shape

    bits = max((VOCAB - 1).bit_length(), 1)
    shifts = jnp.asarray([1 << (bits * s) for s in range(SEQ)], jnp.int32)
    tok_packed = (tokens * shifts).sum(axis=1, dtype=jnp.int32)
    tt = jnp.pad(table.T, ((0, 1), (0, VP - VOCAB)),
                 constant_values=1.0)
    wb = jnp.concatenate([W.reshape(EMB), b]).reshape(EMB + 1, 1)
    wb = jnp.broadcast_to(wb, (EMB + 1, L))
    wt = jnp.concatenate([tt, wb], axis=1).reshape(-1)
    packed = jnp.concatenate([tok_packed,
                              lax.bitcast_convert_type(wt, jnp.int32)])

    out = _make_kernel(B, SEQ, EMB + 1, bits)(packed)
    return out.reshape(B, 1)
